# baseline (speedup 1.0000x reference)
import functools

import jax
import jax.numpy as jnp
from jax.experimental import pallas as pl

_TOPK = 8
_NEXP = 64


def _gate_block(x_ref, w_ref, wout_ref, iout_ref):
    x = x_ref[...]
    w = w_ref[...]
    scores = jax.lax.dot_general(
        x, w, (((1,), (1,)), ((), ())), preferred_element_type=jnp.float32
    )
    scores = jax.nn.sigmoid(scores)
    rows = scores.shape[0]
    iota = jax.lax.broadcasted_iota(jnp.int32, (rows, _NEXP), 1)
    work = scores
    vals = []
    idxs = []
    for _ in range(_TOPK):
        m = jnp.max(work, axis=1, keepdims=True)
        cand = jnp.where(work == m, iota, _NEXP)
        idx = jnp.min(cand, axis=1, keepdims=True)
        vals.append(m)
        idxs.append(idx)
        work = jnp.where(iota == idx, -jnp.inf, work)
    total = vals[0]
    for v in vals[1:]:
        total = total + v
    wout_ref[...] = jnp.concatenate(vals, axis=1) / total
    iout_ref[...] = jnp.concatenate(idxs, axis=1)


@jax.jit
def kernel(x, W):
    tokens = x.shape[0]
    block = 512
    grid = tokens // block
    wout, iout = pl.pallas_call(
        _gate_block,
        grid=(grid,),
        in_specs=[
            pl.BlockSpec((block, x.shape[1]), lambda i: (i, 0)),
            pl.BlockSpec((_NEXP, x.shape[1]), lambda i: (0, 0)),
        ],
        out_specs=[
            pl.BlockSpec((block, _TOPK), lambda i: (i, 0)),
            pl.BlockSpec((block, _TOPK), lambda i: (i, 0)),
        ],
        out_shape=[
            jax.ShapeDtypeStruct((tokens, _TOPK), jnp.float32),
            jax.ShapeDtypeStruct((tokens, _TOPK), jnp.int32),
        ],
    )(x, W)
    return (wout, iout)


# speedup vs baseline: 1.6535x; 1.6535x over previous
import functools

import jax
import jax.numpy as jnp
from jax.experimental import pallas as pl

_TOPK = 8
_NEXP = 64


def _gate_block(x_ref, w_ref, wout_ref, iout_ref):
    x = x_ref[...]
    w = w_ref[...]
    st = jax.lax.dot_general(
        w, x, (((1,), (1,)), ((), ())), preferred_element_type=jnp.float32
    )
    st = jax.nn.sigmoid(st)
    iota = jax.lax.broadcasted_iota(jnp.int32, st.shape, 0)
    work = st
    vals = []
    idxs = []
    for _ in range(_TOPK):
        m = jnp.max(work, axis=0, keepdims=True)
        cand = jnp.where(work == m, iota, _NEXP)
        idx = jnp.min(cand, axis=0, keepdims=True)
        vals.append(m)
        idxs.append(idx)
        work = jnp.where(iota == idx, -jnp.inf, work)
    total = vals[0]
    for v in vals[1:]:
        total = total + v
    wt = jnp.concatenate(vals, axis=0) / total
    it = jnp.concatenate(idxs, axis=0)
    wout_ref[...] = wt.T
    iout_ref[...] = it.T


@jax.jit
def kernel(x, W):
    tokens = x.shape[0]
    block = 1024
    grid = tokens // block
    wout, iout = pl.pallas_call(
        _gate_block,
        grid=(grid,),
        in_specs=[
            pl.BlockSpec((block, x.shape[1]), lambda i: (i, 0)),
            pl.BlockSpec((_NEXP, x.shape[1]), lambda i: (0, 0)),
        ],
        out_specs=[
            pl.BlockSpec((block, _TOPK), lambda i: (i, 0)),
            pl.BlockSpec((block, _TOPK), lambda i: (i, 0)),
        ],
        out_shape=[
            jax.ShapeDtypeStruct((tokens, _TOPK), jnp.float32),
            jax.ShapeDtypeStruct((tokens, _TOPK), jnp.int32),
        ],
    )(x, W)
    return (wout, iout)


# speedup vs baseline: 1.7393x; 1.0519x over previous
import functools

import jax
import jax.numpy as jnp
from jax.experimental import pallas as pl

_TOPK = 8
_NEXP = 64


def _gate_block(x_ref, w_ref, wout_ref, iout_ref):
    x = x_ref[...]
    w = w_ref[...]
    st = jax.lax.dot_general(
        w, x, (((1,), (1,)), ((), ())), preferred_element_type=jnp.float32
    )
    st = jax.nn.sigmoid(st)
    iota = jax.lax.broadcasted_iota(jnp.int32, st.shape, 0)
    work = st
    vals = []
    idxs = []
    for _ in range(_TOPK):
        m = jnp.max(work, axis=0, keepdims=True)
        cand = jnp.where(work == m, iota, _NEXP)
        idx = jnp.min(cand, axis=0, keepdims=True)
        vals.append(m)
        idxs.append(idx)
        work = jnp.where(iota == idx, -jnp.inf, work)
    total = vals[0]
    for v in vals[1:]:
        total = total + v
    wt = jnp.concatenate(vals, axis=0) / total
    it = jnp.concatenate(idxs, axis=0)
    wout_ref[...] = wt.T
    iout_ref[...] = it.T


@jax.jit
def kernel(x, W):
    tokens = x.shape[0]
    block = 2048
    grid = tokens // block
    wout, iout = pl.pallas_call(
        _gate_block,
        grid=(grid,),
        in_specs=[
            pl.BlockSpec((block, x.shape[1]), lambda i: (i, 0)),
            pl.BlockSpec((_NEXP, x.shape[1]), lambda i: (0, 0)),
        ],
        out_specs=[
            pl.BlockSpec((block, _TOPK), lambda i: (i, 0)),
            pl.BlockSpec((block, _TOPK), lambda i: (i, 0)),
        ],
        out_shape=[
            jax.ShapeDtypeStruct((tokens, _TOPK), jnp.float32),
            jax.ShapeDtypeStruct((tokens, _TOPK), jnp.int32),
        ],
    )(x, W)
    return (wout, iout)
